# Initial kernel scaffold; baseline (speedup 1.0000x reference)
import jax
import jax.numpy as jnp
from jax.experimental import pallas as pl


def kernel(atom_in_fea, nbr_fea, nbr_fea_idx, bond_weights_i, bond_weights_j, W_full, b_full, bn1_gamma, bn1_beta, bn2_gamma, bn2_beta):
    raise NotImplementedError("write your pallas kernel here")


# speedup vs baseline: 1.4060x; 1.4060x over previous
import functools

import jax
import jax.numpy as jnp
from jax import lax
from jax.experimental import pallas as pl
from jax.experimental.pallas import tpu as pltpu
from jax.experimental.pallas import tpu_sc as plsc

N = 10000
M = 32
D = 128
K = 16
NW = 32
C = 320
NPAD = NW * C
SPC = 4
EPS = SPC * M
NSTEP = C // SPC


def _sc_gather_kernel(atom_hbm, idx_hbm, bwi_hbm, bwj_hbm, nbr_hbm,
                      g_hbm, f_hbm,
                      idx_v, bwi_v, bwj_v, rows_v, nbr_v,
                      g_v, f_v, sem, nsem):
    nc = 2
    wid = lax.axis_index("s") * nc + lax.axis_index("c")
    ebase = wid * (C * M)
    cbase = wid * C

    pltpu.sync_copy(idx_hbm.at[pl.ds(ebase, C * M)], idx_v)
    pltpu.sync_copy(bwi_hbm.at[pl.ds(ebase, C * M)], bwi_v)
    pltpu.sync_copy(bwj_hbm.at[pl.ds(ebase, C * M)], bwj_v)

    def step(g, _):
        e0 = g * EPS
        cp = pltpu.async_copy(atom_hbm.at[idx_v.at[pl.ds(e0, EPS)]], rows_v, sem)
        cpn = pltpu.async_copy(nbr_hbm.at[pl.ds((ebase + e0) * K, EPS * K)],
                               nbr_v, nsem)
        cp.wait()
        cpn.wait()
        for ci in range(SPC):
            c = g * SPC + ci
            wv = []
            for h in range(M // 16):
                wi = bwi_v[pl.ds(e0 + ci * M + h * 16, 16)]
                wj = bwj_v[pl.ds(e0 + ci * M + h * 16, 16)]
                wv.append(wi * wj)
            f_acc = jnp.zeros((K,), jnp.float32)
            g_acc = [jnp.zeros((16,), jnp.float32) for _ in range(D // 16)]
            for m in range(M):
                e = ci * M + m
                w = wv[m // 16][m % 16]
                f_acc = f_acc + w * nbr_v[pl.ds(e * K, K)]
                for k in range(D // 16):
                    g_acc[k] = g_acc[k] + w * rows_v[e, pl.ds(k * 16, 16)]
            f_v[pl.ds(c * K, K)] = f_acc
            for k in range(D // 16):
                g_v[pl.ds(c * D + k * 16, 16)] = g_acc[k]
        return _

    lax.fori_loop(0, NSTEP, step, 0)

    pltpu.sync_copy(g_v, g_hbm.at[pl.ds(cbase * D, C * D)])
    pltpu.sync_copy(f_v, f_hbm.at[pl.ds(cbase * K, C * K)])


@jax.jit
def _sc_gather(atom_in_fea, idx_flat, bwi_flat, bwj_flat, nbr_flat):
    mesh = plsc.VectorSubcoreMesh(core_axis_name="c", subcore_axis_name="s")
    f = pl.kernel(
        _sc_gather_kernel,
        out_type=[
            jax.ShapeDtypeStruct((NPAD * D,), jnp.float32),
            jax.ShapeDtypeStruct((NPAD * K,), jnp.float32),
        ],
        mesh=mesh,
        scratch_types=[
            pltpu.VMEM((C * M,), jnp.int32),
            pltpu.VMEM((C * M,), jnp.float32),
            pltpu.VMEM((C * M,), jnp.float32),
            pltpu.VMEM((EPS, D), jnp.float32),
            pltpu.VMEM((EPS * K,), jnp.float32),
            pltpu.VMEM((C * D,), jnp.float32),
            pltpu.VMEM((C * K,), jnp.float32),
            pltpu.SemaphoreType.DMA,
            pltpu.SemaphoreType.DMA,
        ],
    )
    return f(atom_in_fea, idx_flat, bwi_flat, bwj_flat, nbr_flat)


def _tc_tail_kernel(atom_ref, bwi_ref, bwj_ref, g_ref, f_ref, wc_ref, wn_ref,
                    wf_ref, b_ref, g1_ref, b1_ref, g2_ref, b2_ref, out_ref):
    atom = atom_ref[...]
    s = jnp.sum(bwi_ref[...] * bwj_ref[...], axis=1, keepdims=True)
    z = jnp.dot(atom * s, wc_ref[...], preferred_element_type=jnp.float32)
    z = z + jnp.dot(g_ref[...], wn_ref[...], preferred_element_type=jnp.float32)
    z = z + jnp.dot(f_ref[...], wf_ref[...], preferred_element_type=jnp.float32)
    z = z + b_ref[...]

    mean1 = jnp.mean(z, axis=0, keepdims=True)
    zc = z - mean1
    var1 = jnp.mean(zc * zc, axis=0, keepdims=True)
    zn = zc * lax.rsqrt(var1 + 1e-5) * g1_ref[...] + b1_ref[...]

    filt = zn[:, :D]
    core = zn[:, D:]
    a = (1.0 / (1.0 + jnp.exp(-filt))) * (
        jnp.maximum(core, 0.0) + jnp.log1p(jnp.exp(-jnp.abs(core))))

    mean2 = jnp.mean(a, axis=0, keepdims=True)
    ac = a - mean2
    var2 = jnp.mean(ac * ac, axis=0, keepdims=True)
    an = ac * lax.rsqrt(var2 + 1e-5) * g2_ref[...] + b2_ref[...]
    out_ref[...] = jnp.maximum(an, 0.0) + jnp.log1p(jnp.exp(-jnp.abs(an)))


@jax.jit
def _tc_tail(atom_in_fea, bwi, bwj, G, F, WcT, WnT, WfT, b, g1, b1, g2, b2):
    return pl.pallas_call(
        _tc_tail_kernel,
        out_shape=jax.ShapeDtypeStruct((N, D), jnp.float32),
    )(atom_in_fea, bwi, bwj, G, F, WcT, WnT, WfT, b, g1, b1, g2, b2)


def kernel(atom_in_fea, nbr_fea, nbr_fea_idx, bond_weights_i, bond_weights_j,
           W_full, b_full, bn1_gamma, bn1_beta, bn2_gamma, bn2_beta):
    pad = NPAD - N
    idx_flat = jnp.pad(nbr_fea_idx.reshape(-1), (0, pad * M))
    bwi_flat = jnp.pad(bond_weights_i.reshape(-1), (0, pad * M))
    bwj_flat = jnp.pad(bond_weights_j.reshape(-1), (0, pad * M))
    nbr_flat = jnp.pad(nbr_fea.reshape(-1), (0, pad * M * K))

    G, F = _sc_gather(atom_in_fea, idx_flat, bwi_flat, bwj_flat, nbr_flat)
    G = G.reshape(NPAD, D)
    F = F.reshape(NPAD, K)

    WT = W_full.T
    WcT = WT[:D]
    WnT = WT[D:2 * D]
    WfT = WT[2 * D:]
    return _tc_tail(atom_in_fea, bond_weights_i, bond_weights_j, G[:N], F[:N],
                    WcT, WnT, WfT, b_full[None, :],
                    bn1_gamma[None, :], bn1_beta[None, :],
                    bn2_gamma[None, :], bn2_beta[None, :])


# speedup vs baseline: 1.7016x; 1.2103x over previous
import jax
import jax.numpy as jnp
from jax import lax
from jax.experimental import pallas as pl
from jax.experimental.pallas import tpu as pltpu
from jax.experimental.pallas import tpu_sc as plsc

N = 10000
M = 32
D = 128
K = 16
NW = 32
C = 320
NPAD = NW * C
SPC = 4
EPS = SPC * M
NSTEP = C // SPC
NB = 4
NEDGE = N * M


def _sc_gather_kernel(atom_hbm, idx_hbm, bwi_hbm, bwj_hbm, nbr_hbm,
                      g_hbm, f_hbm,
                      idx_v, bwi_v, bwj_v,
                      rows0, rows1, rows2, rows3,
                      nbr0, nbr1, nbr2, nbr3,
                      go0, go1, go2, go3,
                      fo0, fo1, fo2, fo3,
                      gsem, nsem, osem, fsem):
    rows = [rows0, rows1, rows2, rows3]
    nbr = [nbr0, nbr1, nbr2, nbr3]
    go = [go0, go1, go2, go3]
    fo = [fo0, fo1, fo2, fo3]

    nc = 2
    wid = lax.axis_index("s") * nc + lax.axis_index("c")
    ebase = wid * (C * M)
    cbase = wid * C

    pltpu.sync_copy(idx_hbm.at[pl.ds(ebase, C * M)], idx_v)
    pltpu.sync_copy(bwi_hbm.at[pl.ds(ebase, C * M)], bwi_v)
    pltpu.sync_copy(bwj_hbm.at[pl.ds(ebase, C * M)], bwj_v)

    def issue(g, b):
        e0 = g * EPS
        pltpu.async_copy(atom_hbm.at[idx_v.at[pl.ds(e0, EPS)]], rows[b], gsem)
        src = jnp.minimum(ebase + e0, NEDGE - EPS) * K
        pltpu.async_copy(nbr_hbm.at[pl.ds(src, EPS * K)], nbr[b], nsem)

    def compute(g, b):
        e0 = g * EPS
        for ci in range(SPC):
            wv = []
            for h in range(M // 16):
                wi = bwi_v[pl.ds(e0 + ci * M + h * 16, 16)]
                wj = bwj_v[pl.ds(e0 + ci * M + h * 16, 16)]
                wv.append(wi * wj)
            f_acc = jnp.zeros((K,), jnp.float32)
            g_acc = [jnp.zeros((16,), jnp.float32) for _ in range(D // 16)]
            for m in range(M):
                e = ci * M + m
                w = wv[m // 16][m % 16]
                f_acc = f_acc + w * nbr[b][pl.ds(e * K, K)]
                for k in range(D // 16):
                    g_acc[k] = g_acc[k] + w * rows[b][e, pl.ds(k * 16, 16)]
            fo[b][pl.ds(ci * K, K)] = f_acc
            for k in range(D // 16):
                go[b][pl.ds(ci * D + k * 16, 16)] = g_acc[k]

    for b in range(NB):
        issue(b, b)

    def outer(t, carry):
        g0 = t * NB
        for b in range(NB):
            g = g0 + b
            pltpu.make_async_copy(
                atom_hbm.at[idx_v.at[pl.ds(0, EPS)]], rows[b], gsem).wait()
            pltpu.make_async_copy(
                nbr_hbm.at[pl.ds(0, EPS * K)], nbr[b], nsem).wait()

            @pl.when(t > 0)
            def _wait_out():
                pltpu.make_async_copy(
                    go[b], g_hbm.at[pl.ds(0, SPC * D)], osem).wait()
                pltpu.make_async_copy(
                    fo[b], f_hbm.at[pl.ds(0, SPC * K)], fsem).wait()

            compute(g, b)
            pltpu.async_copy(
                go[b], g_hbm.at[pl.ds((cbase + g * SPC) * D, SPC * D)], osem)
            pltpu.async_copy(
                fo[b], f_hbm.at[pl.ds((cbase + g * SPC) * K, SPC * K)], fsem)

            nxt = g + NB

            @pl.when(nxt < NSTEP)
            def _issue_next():
                issue(nxt, b)
        return carry

    lax.fori_loop(0, NSTEP // NB, outer, 0)

    for b in range(NB):
        pltpu.make_async_copy(go[b], g_hbm.at[pl.ds(0, SPC * D)], osem).wait()
        pltpu.make_async_copy(fo[b], f_hbm.at[pl.ds(0, SPC * K)], fsem).wait()


@jax.jit
def _sc_gather(atom_in_fea, idx_flat, bwi_flat, bwj_flat, nbr_flat):
    mesh = plsc.VectorSubcoreMesh(core_axis_name="c", subcore_axis_name="s")
    f = pl.kernel(
        _sc_gather_kernel,
        out_type=[
            jax.ShapeDtypeStruct((NPAD * D,), jnp.float32),
            jax.ShapeDtypeStruct((NPAD * K,), jnp.float32),
        ],
        mesh=mesh,
        scratch_types=(
            [
                pltpu.VMEM((C * M,), jnp.int32),
                pltpu.VMEM((C * M,), jnp.float32),
                pltpu.VMEM((C * M,), jnp.float32),
            ]
            + [pltpu.VMEM((EPS, D), jnp.float32)] * NB
            + [pltpu.VMEM((EPS * K,), jnp.float32)] * NB
            + [pltpu.VMEM((SPC * D,), jnp.float32)] * NB
            + [pltpu.VMEM((SPC * K,), jnp.float32)] * NB
            + [pltpu.SemaphoreType.DMA] * 4
        ),
    )
    return f(atom_in_fea, idx_flat, bwi_flat, bwj_flat, nbr_flat)


def _tc_tail_kernel(atom_ref, bwi_ref, bwj_ref, g_ref, f_ref, wc_ref, wn_ref,
                    wf_ref, b_ref, g1_ref, b1_ref, g2_ref, b2_ref, out_ref):
    atom = atom_ref[...]
    s = jnp.sum(bwi_ref[...] * bwj_ref[...], axis=1, keepdims=True)
    z = jnp.dot(atom * s, wc_ref[...], preferred_element_type=jnp.float32)
    z = z + jnp.dot(g_ref[...][:N], wn_ref[...],
                    preferred_element_type=jnp.float32)
    z = z + jnp.dot(f_ref[...][:N], wf_ref[...],
                    preferred_element_type=jnp.float32)
    z = z + b_ref[...]

    mean1 = jnp.mean(z, axis=0, keepdims=True)
    zc = z - mean1
    var1 = jnp.mean(zc * zc, axis=0, keepdims=True)
    zn = zc * lax.rsqrt(var1 + 1e-5) * g1_ref[...] + b1_ref[...]

    filt = zn[:, :D]
    core = zn[:, D:]
    a = (1.0 / (1.0 + jnp.exp(-filt))) * (
        jnp.maximum(core, 0.0) + jnp.log1p(jnp.exp(-jnp.abs(core))))

    mean2 = jnp.mean(a, axis=0, keepdims=True)
    ac = a - mean2
    var2 = jnp.mean(ac * ac, axis=0, keepdims=True)
    an = ac * lax.rsqrt(var2 + 1e-5) * g2_ref[...] + b2_ref[...]
    out_ref[...] = jnp.maximum(an, 0.0) + jnp.log1p(jnp.exp(-jnp.abs(an)))


@jax.jit
def _tc_tail(atom_in_fea, bwi, bwj, G, F, WcT, WnT, WfT, b, g1, b1, g2, b2):
    return pl.pallas_call(
        _tc_tail_kernel,
        out_shape=jax.ShapeDtypeStruct((N, D), jnp.float32),
    )(atom_in_fea, bwi, bwj, G, F, WcT, WnT, WfT, b, g1, b1, g2, b2)


def kernel(atom_in_fea, nbr_fea, nbr_fea_idx, bond_weights_i, bond_weights_j,
           W_full, b_full, bn1_gamma, bn1_beta, bn2_gamma, bn2_beta):
    pad = NPAD - N
    idx_flat = jnp.pad(nbr_fea_idx.reshape(-1), (0, pad * M))
    bwi_flat = jnp.pad(bond_weights_i.reshape(-1), (0, pad * M))
    bwj_flat = jnp.pad(bond_weights_j.reshape(-1), (0, pad * M))
    nbr_flat = nbr_fea.reshape(-1)

    G, F = _sc_gather(atom_in_fea, idx_flat, bwi_flat, bwj_flat, nbr_flat)
    G = G.reshape(NPAD, D)
    F = F.reshape(NPAD, K)

    WT = W_full.T
    WcT = WT[:D]
    WnT = WT[D:2 * D]
    WfT = WT[2 * D:]
    return _tc_tail(atom_in_fea, bond_weights_i, bond_weights_j, G, F,
                    WcT, WnT, WfT, b_full[None, :],
                    bn1_gamma[None, :], bn1_beta[None, :],
                    bn2_gamma[None, :], bn2_beta[None, :])


# speedup vs baseline: 3.7595x; 2.2094x over previous
import jax
import jax.numpy as jnp
from jax import lax
from jax.experimental import pallas as pl
from jax.experimental.pallas import tpu as pltpu
from jax.experimental.pallas import tpu_sc as plsc

N = 10000
M = 32
D = 128
K = 16
NW = 32
C = 320
NPAD = NW * C
SPC = 4
EPS = SPC * M
NSTEP = C // SPC
NB = 2
MD = 4
NEDGE = N * M


def _sc_gather_kernel(atom_hbm, idx_hbm, bwi_hbm, bwj_hbm, nbr_hbm,
                      g_hbm, f_hbm,
                      table_sp,
                      ix0, ix1, ix2, ix3,
                      bi0, bi1, bi2, bi3,
                      bj0, bj1, bj2, bj3,
                      rows0, rows1, nbr0, nbr1, go0, go1, fo0, fo1,
                      gsem, nsem, osem, fsem, msem):
    ix = [ix0, ix1, ix2, ix3]
    bi = [bi0, bi1, bi2, bi3]
    bj = [bj0, bj1, bj2, bj3]
    rows = [rows0, rows1]
    nbr = [nbr0, nbr1]
    go = [go0, go1]
    fo = [fo0, fo1]

    nc = 2
    wid = lax.axis_index("s") * nc + lax.axis_index("c")
    ebase = wid * (C * M)
    cbase = wid * C

    @pl.when(lax.axis_index("s") == 0)
    def _stage_table():
        pltpu.sync_copy(atom_hbm, table_sp)

    def meta_issue(g, p):
        e0 = ebase + g * EPS
        pltpu.async_copy(idx_hbm.at[pl.ds(e0, EPS)], ix[p], msem)
        pltpu.async_copy(bwi_hbm.at[pl.ds(e0, EPS)], bi[p], msem)
        pltpu.async_copy(bwj_hbm.at[pl.ds(e0, EPS)], bj[p], msem)

    def meta_wait(p):
        pltpu.make_async_copy(idx_hbm.at[pl.ds(0, EPS)], ix[p], msem).wait()
        pltpu.make_async_copy(bwi_hbm.at[pl.ds(0, EPS)], bi[p], msem).wait()
        pltpu.make_async_copy(bwj_hbm.at[pl.ds(0, EPS)], bj[p], msem).wait()

    def issue(g, p, rb):
        pltpu.async_copy(table_sp.at[ix[p]], rows[rb], gsem)
        src = jnp.minimum(ebase + g * EPS, NEDGE - EPS) * K
        pltpu.async_copy(nbr_hbm.at[pl.ds(src, EPS * K)], nbr[rb], nsem)

    def compute(p, rb):
        def center(ci, carry):
            wv = []
            for h in range(M // 16):
                wiv = bi[p][pl.ds(ci * M + h * 16, 16)]
                wjv = bj[p][pl.ds(ci * M + h * 16, 16)]
                wv.append(wiv * wjv)
            f_acc = jnp.zeros((K,), jnp.float32)
            g_acc = [jnp.zeros((16,), jnp.float32) for _ in range(D // 16)]
            for m in range(M):
                w = wv[m // 16][m % 16]
                f_acc = f_acc + w * nbr[rb][pl.ds((ci * M + m) * K, K)]
                for k in range(D // 16):
                    g_acc[k] = g_acc[k] + w * rows[rb][ci * M + m,
                                                       pl.ds(k * 16, 16)]
            fo[rb][pl.ds(ci * K, K)] = f_acc
            for k in range(D // 16):
                go[rb][pl.ds(ci * D + k * 16, 16)] = g_acc[k]
            return carry

        lax.fori_loop(0, SPC, center, 0)

    for p in range(MD):
        meta_issue(p, p)
    plsc.subcore_barrier()
    meta_wait(0)
    issue(0, 0, 0)
    meta_wait(1)
    issue(1, 1, 1)

    def outer(t, carry):
        g0 = t * MD
        for b in range(MD):
            g = g0 + b
            rb = b % NB
            pltpu.make_async_copy(
                table_sp.at[ix[0]], rows[rb], gsem).wait()
            pltpu.make_async_copy(
                nbr_hbm.at[pl.ds(0, EPS * K)], nbr[rb], nsem).wait()

            @pl.when(g >= NB)
            def _wait_out():
                pltpu.make_async_copy(
                    go[rb], g_hbm.at[pl.ds(0, SPC * D)], osem).wait()
                pltpu.make_async_copy(
                    fo[rb], f_hbm.at[pl.ds(0, SPC * K)], fsem).wait()

            compute(b, rb)
            pltpu.async_copy(
                go[rb], g_hbm.at[pl.ds((cbase + g * SPC) * D, SPC * D)], osem)
            pltpu.async_copy(
                fo[rb], f_hbm.at[pl.ds((cbase + g * SPC) * K, SPC * K)], fsem)

            @pl.when(g + MD < NSTEP)
            def _issue_meta():
                meta_issue(g + MD, b)

            @pl.when(g + NB < NSTEP)
            def _issue_gather():
                meta_wait((b + NB) % MD)
                issue(g + NB, (b + NB) % MD, rb)
        return carry

    lax.fori_loop(0, NSTEP // MD, outer, 0)

    for rb in range(NB):
        pltpu.make_async_copy(go[rb], g_hbm.at[pl.ds(0, SPC * D)], osem).wait()
        pltpu.make_async_copy(fo[rb], f_hbm.at[pl.ds(0, SPC * K)], fsem).wait()


@jax.jit
def _sc_gather(atom_in_fea, idx_flat, bwi_flat, bwj_flat, nbr_flat):
    mesh = plsc.VectorSubcoreMesh(core_axis_name="c", subcore_axis_name="s")
    f = pl.kernel(
        _sc_gather_kernel,
        out_type=[
            jax.ShapeDtypeStruct((NPAD * D,), jnp.float32),
            jax.ShapeDtypeStruct((NPAD * K,), jnp.float32),
        ],
        mesh=mesh,
        scratch_types=(
            [pltpu.VMEM_SHARED((N, D), jnp.float32)]
            + [pltpu.VMEM((EPS,), jnp.int32)] * MD
            + [pltpu.VMEM((EPS,), jnp.float32)] * MD
            + [pltpu.VMEM((EPS,), jnp.float32)] * MD
            + [pltpu.VMEM((EPS, D), jnp.float32)] * NB
            + [pltpu.VMEM((EPS * K,), jnp.float32)] * NB
            + [pltpu.VMEM((SPC * D,), jnp.float32)] * NB
            + [pltpu.VMEM((SPC * K,), jnp.float32)] * NB
            + [pltpu.SemaphoreType.DMA] * 5
        ),
    )
    return f(atom_in_fea, idx_flat, bwi_flat, bwj_flat, nbr_flat)


def _tc_tail_kernel(atom_ref, bwi_ref, bwj_ref, g_ref, f_ref, wc_ref, wn_ref,
                    wf_ref, b_ref, g1_ref, b1_ref, g2_ref, b2_ref, out_ref):
    atom = atom_ref[...]
    s = jnp.sum(bwi_ref[...] * bwj_ref[...], axis=1, keepdims=True)
    z = jnp.dot(atom * s, wc_ref[...], preferred_element_type=jnp.float32)
    z = z + jnp.dot(g_ref[...][:N], wn_ref[...],
                    preferred_element_type=jnp.float32)
    z = z + jnp.dot(f_ref[...][:N], wf_ref[...],
                    preferred_element_type=jnp.float32)
    z = z + b_ref[...]

    mean1 = jnp.mean(z, axis=0, keepdims=True)
    zc = z - mean1
    var1 = jnp.mean(zc * zc, axis=0, keepdims=True)
    zn = zc * lax.rsqrt(var1 + 1e-5) * g1_ref[...] + b1_ref[...]

    filt = zn[:, :D]
    core = zn[:, D:]
    a = (1.0 / (1.0 + jnp.exp(-filt))) * (
        jnp.maximum(core, 0.0) + jnp.log1p(jnp.exp(-jnp.abs(core))))

    mean2 = jnp.mean(a, axis=0, keepdims=True)
    ac = a - mean2
    var2 = jnp.mean(ac * ac, axis=0, keepdims=True)
    an = ac * lax.rsqrt(var2 + 1e-5) * g2_ref[...] + b2_ref[...]
    out_ref[...] = jnp.maximum(an, 0.0) + jnp.log1p(jnp.exp(-jnp.abs(an)))


@jax.jit
def _tc_tail(atom_in_fea, bwi, bwj, G, F, WcT, WnT, WfT, b, g1, b1, g2, b2):
    return pl.pallas_call(
        _tc_tail_kernel,
        out_shape=jax.ShapeDtypeStruct((N, D), jnp.float32),
    )(atom_in_fea, bwi, bwj, G, F, WcT, WnT, WfT, b, g1, b1, g2, b2)


def kernel(atom_in_fea, nbr_fea, nbr_fea_idx, bond_weights_i, bond_weights_j,
           W_full, b_full, bn1_gamma, bn1_beta, bn2_gamma, bn2_beta):
    pad = NPAD - N
    idx_flat = jnp.pad(nbr_fea_idx.reshape(-1), (0, pad * M))
    bwi_flat = jnp.pad(bond_weights_i.reshape(-1), (0, pad * M))
    bwj_flat = jnp.pad(bond_weights_j.reshape(-1), (0, pad * M))
    nbr_flat = nbr_fea.reshape(-1)

    G, F = _sc_gather(atom_in_fea, idx_flat, bwi_flat, bwj_flat, nbr_flat)
    G = G.reshape(NPAD, D)
    F = F.reshape(NPAD, K)

    WT = W_full.T
    WcT = WT[:D]
    WnT = WT[D:2 * D]
    WfT = WT[2 * D:]
    return _tc_tail(atom_in_fea, bond_weights_i, bond_weights_j, G, F,
                    WcT, WnT, WfT, b_full[None, :],
                    bn1_gamma[None, :], bn1_beta[None, :],
                    bn2_gamma[None, :], bn2_beta[None, :])
